# baseline (speedup 1.0000x reference)
import functools

import jax
import jax.numpy as jnp
from jax import lax
from jax.experimental import pallas as pl
from jax.experimental.pallas import tpu as pltpu
from jax.experimental.pallas import tpu_sc as plsc

D = 512

_mesh = plsc.VectorSubcoreMesh(
    core_axis_name="c", subcore_axis_name="s", num_cores=1, num_subcores=1
)


@functools.partial(
    pl.kernel,
    mesh=_mesh,
    out_type=(
        jax.ShapeDtypeStruct((1, 3), jnp.float32),
        jax.ShapeDtypeStruct((1, 1), jnp.float32),
        jax.ShapeDtypeStruct((1, 2), jnp.int32),
    ),
    scratch_types=[
        pltpu.VMEM((16,), jnp.int32),
        pltpu.VMEM((16,), jnp.float32),
        pltpu.VMEM((16,), jnp.float32),
        pltpu.VMEM((16,), jnp.int32),
        pltpu.SemaphoreType.DMA,
    ],
    compiler_params=pltpu.CompilerParams(needs_layout_passes=False),
)
def _aux_sc(ev_hbm, nodes_hbm, feat_hbm, edges_hbm,
            ev_v, aux_v, feat_v, zed_v, sem_ev):
    pltpu.async_copy(ev_hbm, ev_v, sem_ev).wait()
    lane = lax.iota(jnp.int32, 16)
    ev = ev_v[...]
    evf = ev.astype(jnp.float32)
    f_s = jnp.sum(jnp.where(lane == 3, evf, jnp.zeros((16,), jnp.float32)))
    aux_v[...] = evf
    feat_v[...] = jnp.zeros((16,), jnp.float32) + f_s
    zed_v[...] = jnp.zeros((16,), jnp.int32)
    i0 = jnp.int32(0)
    pltpu.sync_copy(aux_v.at[pl.ds(0, 3)], nodes_hbm.at[i0])
    pltpu.sync_copy(feat_v.at[pl.ds(0, 1)], feat_hbm.at[i0])
    pltpu.sync_copy(zed_v.at[pl.ds(0, 2)], edges_hbm.at[i0])


def _tc_body(ev_ref, mat_ref, out_ref):
    x = ev_ref[0]
    y = ev_ref[1]
    r = lax.broadcasted_iota(jnp.int32, (D, D), 0)
    c = lax.broadcasted_iota(jnp.int32, (D, D), 1)
    out_ref[...] = jnp.where((r == x) & (c == y), 0, mat_ref[...])


_tc_call = pl.pallas_call(
    _tc_body,
    in_specs=[
        pl.BlockSpec(memory_space=pltpu.SMEM),
        pl.BlockSpec(memory_space=pltpu.VMEM),
    ],
    out_specs=pl.BlockSpec(memory_space=pltpu.VMEM),
    out_shape=jax.ShapeDtypeStruct((D, D), jnp.int32),
)


def kernel(event, neighbour_matrix):
    ev = event.astype(jnp.int32)
    ev16 = jnp.zeros((16,), jnp.int32).at[:4].set(ev)
    nodes, features, edges = _aux_sc(ev16)
    new_matrix = _tc_call(ev, neighbour_matrix)
    return nodes, features, edges, new_matrix
